# Initial kernel scaffold; baseline (speedup 1.0000x reference)
import jax
import jax.numpy as jnp
from jax.experimental import pallas as pl


def kernel(x_artist, x_track, x_tag, edge_attr_lastfm, params, edge_index_collab, edge_index_lastfm, edge_index_linked, edge_index_musical, edge_index_personal, edge_index_has_tag_artists, edge_index_has_tag_tracks, edge_index_tags_artists, edge_index_tags_tracks, edge_index_worked_by, edge_index_worked_in):
    raise NotImplementedError("write your pallas kernel here")


# speedup vs baseline: 4.9659x; 4.9659x over previous
import functools

import jax
import jax.numpy as jnp
from jax.experimental import pallas as pl

H, C = 3, 64
HID = 64


def _mm_body(x_ref, w_ref, o_ref, *, act):
    acc = jnp.dot(x_ref[...], w_ref[...], preferred_element_type=jnp.float32)
    if act == "relu":
        acc = jnp.maximum(acc, 0.0)
    o_ref[...] = acc


def _mm(x, w, act="none", block=512):
    n, k = x.shape
    m = w.shape[1]
    npad = (-n) % block
    mpad = (-m) % 128
    xp = jnp.pad(x, ((0, npad), (0, 0))) if npad else x
    wp = jnp.pad(w, ((0, 0), (0, mpad))) if mpad else w
    out = pl.pallas_call(
        functools.partial(_mm_body, act=act),
        grid=((n + npad) // block,),
        in_specs=[
            pl.BlockSpec((block, k), lambda i: (i, 0)),
            pl.BlockSpec((k, m + mpad), lambda i: (0, 0)),
        ],
        out_specs=pl.BlockSpec((block, m + mpad), lambda i: (i, 0)),
        out_shape=jax.ShapeDtypeStruct((n + npad, m + mpad), jnp.float32),
    )(xp, wp)
    return out[:n, :m]


def _seg_softmax(logits, seg, n):
    m = jax.ops.segment_max(logits, seg, num_segments=n)
    m = jnp.where(jnp.isfinite(m), m, 0.0)
    e = jnp.exp(logits - m[seg])
    s = jax.ops.segment_sum(e, seg, num_segments=n)
    return e / (s[seg] + 1e-16)


def _gat(x_src, x_dst, ei, p, n_dst):
    hs = _mm(x_src, p['W_src']).reshape(-1, H, C)
    hd = _mm(x_dst, p['W_dst']).reshape(-1, H, C)
    a_s = (hs * p['att_src']).sum(-1)
    a_d = (hd * p['att_dst']).sum(-1)
    src, dst = ei[0], ei[1]
    loop = jnp.arange(n_dst, dtype=src.dtype)
    src = jnp.concatenate([src, loop])
    dst = jnp.concatenate([dst, loop])
    alpha = jax.nn.leaky_relu(a_s[src] + a_d[dst], 0.2)
    alpha = _seg_softmax(alpha, dst, n_dst)
    hs_f = hs.reshape(-1, H * C)
    out = jax.ops.segment_sum(
        hs_f[src] * jnp.repeat(alpha, C, axis=1), dst, num_segments=n_dst)
    return out.reshape(-1, H, C).mean(axis=1) + p['b']


def _gatv2(x_src, x_dst, ei, ea, p, n_dst):
    hs = (_mm(x_src, p['W_l']) + p['b_l']).reshape(-1, H, C)
    hd = (_mm(x_dst, p['W_r']) + p['b_r']).reshape(-1, H, C)
    src, dst = ei[0], ei[1]
    loop = jnp.arange(n_dst, dtype=src.dtype)
    src = jnp.concatenate([src, loop])
    dst = jnp.concatenate([dst, loop])
    ea_full = jnp.concatenate(
        [ea, jnp.tile(ea.mean(axis=0, keepdims=True), (n_dst, 1))], axis=0)
    he = (ea_full @ p['W_e']).reshape(-1, H, C)
    e = jax.nn.leaky_relu(hs[src] + hd[dst] + he, 0.2)
    alpha = (e * p['att']).sum(-1)
    alpha = _seg_softmax(alpha, dst, n_dst)
    hs_f = hs.reshape(-1, H * C)
    out = jax.ops.segment_sum(
        hs_f[src] * jnp.repeat(alpha, C, axis=1), dst, num_segments=n_dst)
    return out.reshape(-1, H, C).mean(axis=1) + p['b']


def _sage(x_src, x_dst, ei, p, n_dst):
    xs = _mm(x_src, p['Wp'], act="none") + p['bp']
    xs = jax.nn.relu(xs)
    src, dst = ei[0], ei[1]
    s = jax.ops.segment_sum(xs[src], dst, num_segments=n_dst)
    cnt = jax.ops.segment_sum(jnp.ones(src.shape[0], dtype=xs.dtype), dst,
                              num_segments=n_dst)
    aggr = s / jnp.maximum(cnt, 1.0)[:, None]
    out = _mm(aggr, p['W_l']) + p['b_l'] + _mm(x_dst, p['W_r'])
    return out / jnp.maximum(jnp.linalg.norm(out, axis=-1, keepdims=True), 1e-12)


def _hetero_layer(xa, xt, xg, E, ea_lf, P):
    Na, Nt, Ng = xa.shape[0], xt.shape[0], xg.shape[0]
    outs_a = [
        _gat(xa, xa, E['collab'], P['collab'], Na),
        _gatv2(xa, xa, E['lastfm'], ea_lf, P['lastfm'], Na),
        _gat(xa, xa, E['linked'], P['linked'], Na),
        _gat(xa, xa, E['musical'], P['musical'], Na),
        _gat(xa, xa, E['personal'], P['personal'], Na),
        _sage(xg, xa, E['tags_artists'], P['tags_artists'], Na),
        _sage(xt, xa, E['worked_by'], P['worked_by'], Na),
    ]
    outs_g = [
        _sage(xa, xg, E['has_tag_artists'], P['has_tag_artists'], Ng),
        _sage(xt, xg, E['has_tag_tracks'], P['has_tag_tracks'], Ng),
    ]
    outs_t = [
        _sage(xg, xt, E['tags_tracks'], P['tags_tracks'], Nt),
        _sage(xa, xt, E['worked_in'], P['worked_in'], Nt),
    ]
    return (jnp.mean(jnp.stack(outs_a), 0), jnp.mean(jnp.stack(outs_t), 0),
            jnp.mean(jnp.stack(outs_g), 0))


def kernel(x_artist, x_track, x_tag, edge_attr_lastfm, params,
           edge_index_collab, edge_index_lastfm, edge_index_linked,
           edge_index_musical, edge_index_personal,
           edge_index_has_tag_artists, edge_index_has_tag_tracks,
           edge_index_tags_artists, edge_index_tags_tracks,
           edge_index_worked_by, edge_index_worked_in):
    E = {'collab': edge_index_collab, 'lastfm': edge_index_lastfm,
         'linked': edge_index_linked, 'musical': edge_index_musical,
         'personal': edge_index_personal,
         'has_tag_artists': edge_index_has_tag_artists,
         'has_tag_tracks': edge_index_has_tag_tracks,
         'tags_artists': edge_index_tags_artists,
         'tags_tracks': edge_index_tags_tracks,
         'worked_by': edge_index_worked_by,
         'worked_in': edge_index_worked_in}
    a1, t1, g1 = _hetero_layer(x_artist, x_track, x_tag, E, edge_attr_lastfm,
                               params['l1'])
    a2, t2, g2 = _hetero_layer(a1, t1, g1, E, ea_lf=edge_attr_lastfm,
                               P=params['l2'])
    x = jnp.concatenate([a1, a2], axis=-1)
    x = jax.nn.relu(_mm(x, params['lin1_W']) + params['lin1_b'])
    x = _mm(x, params['lin2_W']) + params['lin2_b']
    return x / jnp.maximum(jnp.linalg.norm(x, axis=-1, keepdims=True), 1e-12)


# speedup vs baseline: 16.6482x; 3.3525x over previous
import functools

import jax
import jax.numpy as jnp
from jax import lax
from jax.experimental import pallas as pl
from jax.experimental.pallas import tpu as pltpu
from jax.experimental.pallas import tpu_sc as plsc

H, C = 3, 64
HID = 64

NC, NS, NW = 2, 16, 32
SCH = 8192
ACC_R = SCH + 16
SENT = SCH
KB = 128


_SC_PARAMS = pltpu.CompilerParams(needs_layout_passes=False,
                                  use_tc_tiling_on_sc=False)


def _mesh():
    return plsc.VectorSubcoreMesh(core_axis_name="c", subcore_axis_name="s",
                                  num_cores=NC, num_subcores=NS)


def _f16(x):
    return lax.broadcast(x, (16,))


def _cdiv(a, b):
    return -(-a // b)


def _mm_body(x_ref, w_ref, o_ref, *, act):
    acc = jnp.dot(x_ref[...], w_ref[...], preferred_element_type=jnp.float32)
    if act == "relu":
        acc = jnp.maximum(acc, 0.0)
    o_ref[...] = acc


def _mm(x, w, act="none", block=512):
    n, k = x.shape
    m = w.shape[1]
    npad = (-n) % block
    mpad = (-m) % 128
    xp = jnp.pad(x, ((0, npad), (0, 0))) if npad else x
    wp = jnp.pad(w, ((0, 0), (0, mpad))) if mpad else w
    out = pl.pallas_call(
        functools.partial(_mm_body, act=act),
        grid=((n + npad) // block,),
        in_specs=[
            pl.BlockSpec((block, k), lambda i: (i, 0)),
            pl.BlockSpec((k, m + mpad), lambda i: (0, 0)),
        ],
        out_specs=pl.BlockSpec((block, m + mpad), lambda i: (i, 0)),
        out_shape=jax.ShapeDtypeStruct((n + npad, m + mpad), jnp.float32),
    )(xp, wp)
    return out[:n, :m]


def _bucket(ei, n_dst, ea=None):
    e = ei.shape[1]
    esub = _cdiv(_cdiv(e, NW), 8) * 8
    rs = _cdiv(esub, KB) * KB
    nb = _cdiv(n_dst, SCH)
    nv = esub // 16
    stn = rs + 16
    has_ea = ea is not None

    ins = [jnp.pad(ei[0], (0, NW * esub - e)),
           jnp.pad(ei[1], (0, NW * esub - e))]
    if has_ea:
        ins.append(jnp.pad(ea[:, 0], (0, NW * esub - e)))

    def body(*refs):
        if has_ea:
            (eis_h, eid_h, ea_h, bsrc_h, bdst_h, bea_h, cnt_h,
             src_v, dst_v, eav, st_s, st_d, st_e, cnt_v) = refs
        else:
            (eis_h, eid_h, bsrc_h, bdst_h, cnt_h,
             src_v, dst_v, st_s, st_d, cnt_v) = refs
            ea_h = eav = st_e = None
        wid = lax.axis_index("s") * NC + lax.axis_index("c")
        base = wid * esub
        myc = jnp.minimum(e - base, esub)
        pltpu.sync_copy(eis_h.at[pl.ds(base, esub)], src_v)
        pltpu.sync_copy(eid_h.at[pl.ds(base, esub)], dst_v)
        if has_ea:
            pltpu.sync_copy(ea_h.at[pl.ds(base, esub)], eav)

        def zloop(i, _):
            st_s[pl.ds(i * 16, 16)] = jnp.zeros((16,), jnp.int32)
            st_d[pl.ds(i * 16, 16)] = jnp.zeros((16,), jnp.int32)
            if has_ea:
                st_e[pl.ds(i * 16, 16)] = jnp.zeros((16,), jnp.float32)
            return 0
        lax.fori_loop(0, (stn + 16) // 16, zloop, 0)

        iota = lax.iota(jnp.int32, 16)
        for b in range(nb):
            def scan(v, cur):
                sv = src_v[pl.ds(v * 16, 16)]
                dv = dst_v[pl.ds(v * 16, 16)]
                gi = v * 16 + iota
                m = (gi < myc) & ((dv >> 13) == b)
                mi = m.astype(jnp.int32)
                pos = jnp.where(m, cur + lax.cumsum(mi) - 1, stn + iota)
                plsc.store_scatter(st_s, [pos], sv)
                plsc.store_scatter(st_d, [pos], dv)
                if has_ea:
                    ev = eav[pl.ds(v * 16, 16)]
                    plsc.store_scatter(st_e, [pos], ev)
                return cur + jnp.sum(mi)
            cnt = lax.fori_loop(0, nv, scan, jnp.int32(0))
            cnt_v[...] = _f16(cnt)
            pltpu.sync_copy(cnt_v, cnt_h.at[b * NW + wid])
            roff = (b * NW + wid) * rs

            def dma(i, _):
                pltpu.sync_copy(st_s.at[pl.ds(i * KB, KB)],
                                bsrc_h.at[pl.ds(roff + i * KB, KB)])
                pltpu.sync_copy(st_d.at[pl.ds(i * KB, KB)],
                                bdst_h.at[pl.ds(roff + i * KB, KB)])
                if has_ea:
                    pltpu.sync_copy(st_e.at[pl.ds(i * KB, KB)],
                                    bea_h.at[pl.ds(roff + i * KB, KB)])
                return 0
            lax.fori_loop(0, (cnt + KB - 1) // KB, dma, 0)

    out_type = [jax.ShapeDtypeStruct((nb * NW * rs,), jnp.int32),
                jax.ShapeDtypeStruct((nb * NW * rs,), jnp.int32)]
    if has_ea:
        out_type.append(jax.ShapeDtypeStruct((nb * NW * rs,), jnp.float32))
    out_type.append(jax.ShapeDtypeStruct((nb * NW, 16), jnp.int32))

    scratch = [pltpu.VMEM((esub,), jnp.int32), pltpu.VMEM((esub,), jnp.int32)]
    if has_ea:
        scratch.append(pltpu.VMEM((esub,), jnp.float32))
    scratch += [pltpu.VMEM((stn + 16,), jnp.int32),
                pltpu.VMEM((stn + 16,), jnp.int32)]
    if has_ea:
        scratch.append(pltpu.VMEM((stn + 16,), jnp.float32))
    scratch.append(pltpu.VMEM((16,), jnp.int32))

    res = pl.kernel(body, out_type=tuple(out_type), mesh=_mesh(),
                    scratch_types=tuple(scratch),
                    compiler_params=_SC_PARAMS)(*ins)
    if has_ea:
        bsrc, bdst, bea, cnts = res
        return dict(src=bsrc, dst=bdst, ea=bea, cnts=cnts, nb=nb, rs=rs)
    bsrc, bdst, cnts = res
    return dict(src=bsrc, dst=bdst, ea=None, cnts=cnts, nb=nb, rs=rs)


def _edge_agg(kind, bk, table, f, extra=()):
    nb, rs = bk["nb"], bk["rs"]
    kb = 128 if kind == "sage" else 32
    nstripe_z = ACC_R // NS
    nstripe_o = SCH // NS

    def body(*refs):
        it = iter(refs)
        bsrc_h = next(it); bdst_h = next(it)
        bea_h = next(it) if kind == "gatv2" else None
        cnt_h = next(it)
        tab_h = next(it)
        if kind == "gat":
            asr_h = next(it); adr_h = next(it)
        if kind == "gatv2":
            hd_h = next(it); att_h = next(it); wev_h = next(it)
        sum_h = next(it); ws_h = next(it)
        acc = next(it); sacc = next(it)
        src_i = next(it); dst_i = next(it); ldst = next(it)
        rows = next(it); wbuf = next(it)
        zf = next(it); z16 = next(it)
        cntv = next(it)
        if kind == "gat":
            asr_v = next(it); adr_v = next(it)
        if kind == "gatv2":
            rowd = next(it); eav = next(it); attv = next(it); wev_v = next(it)
        sem = next(it)

        core = lax.axis_index("c")
        tid = lax.axis_index("s")
        iota = lax.iota(jnp.int32, 16)

        def zrow(r, _):
            for k in range(f // 16):
                zf[r, pl.ds(k * 16, 16)] = jnp.zeros((16,), jnp.float32)
            fill = jnp.ones((16,), jnp.float32) if kind == "sage" else \
                jnp.zeros((16,), jnp.float32)
            z16[r, pl.ds(0, 16)] = jnp.zeros((16,), jnp.float32)
            wbuf[r, pl.ds(0, 16)] = fill
            return 0
        lax.fori_loop(0, kb, zrow, 0)
        if kind == "gatv2":
            pltpu.sync_copy(att_h, attv)
            pltpu.sync_copy(wev_h, wev_v)

        for ci in range(_cdiv(nb, NC)):
            cc = ci * NC + core

            @pl.when(cc < nb)
            def _chunk():
                for j in range(nstripe_z // kb):
                    pltpu.sync_copy(zf.at[pl.ds(0, kb)],
                                    acc.at[pl.ds(tid * nstripe_z + j * kb, kb)])
                    pltpu.sync_copy(z16.at[pl.ds(0, kb)],
                                    sacc.at[pl.ds(tid * nstripe_z + j * kb, kb)])
                rem = nstripe_z % kb
                if rem:
                    pltpu.sync_copy(zf.at[pl.ds(0, rem)],
                                    acc.at[pl.ds(tid * nstripe_z + (nstripe_z // kb) * kb, rem)])
                    pltpu.sync_copy(z16.at[pl.ds(0, rem)],
                                    sacc.at[pl.ds(tid * nstripe_z + (nstripe_z // kb) * kb, rem)])
                plsc.subcore_barrier()

                for rg in range(2):
                    w = tid * 2 + rg
                    pltpu.sync_copy(cnt_h.at[cc * NW + w], cntv)
                    cnt = jnp.max(cntv[...])
                    roff0 = (cc * NW + w) * rs

                    def blk(i, _):
                        roff = roff0 + i * kb
                        pltpu.sync_copy(bsrc_h.at[pl.ds(roff, kb)], src_i)
                        pltpu.sync_copy(bdst_h.at[pl.ds(roff, kb)], dst_i)
                        pltpu.async_copy(tab_h.at[src_i], rows, sem).wait()
                        if kind == "gat":
                            pltpu.async_copy(asr_h.at[src_i], asr_v, sem).wait()
                            pltpu.async_copy(adr_h.at[dst_i], adr_v, sem).wait()
                        if kind == "gatv2":
                            pltpu.async_copy(hd_h.at[dst_i], rowd, sem).wait()
                            pltpu.sync_copy(bea_h.at[pl.ds(roff, kb)], eav)
                        for v in range(kb // 16):
                            dv = dst_i[pl.ds(v * 16, 16)]
                            gi = i * kb + v * 16 + iota
                            lv = jnp.where(gi < cnt, dv - cc * SCH, SENT)
                            ldst[pl.ds(v * 16, 16)] = lv

                        if kind == "gat":
                            def wcomp(r, _):
                                av = asr_v[r, pl.ds(0, 16)] + adr_v[r, pl.ds(0, 16)]
                                av = jnp.maximum(av, 0.2 * av)
                                wbuf[r, pl.ds(0, 16)] = jnp.exp(av)
                                return 0
                            lax.fori_loop(0, kb, wcomp, 0)
                        if kind == "gatv2":
                            def wcomp(r, _):
                                ear = plsc.load_gather(eav, [_f16(r)])
                                wv = jnp.zeros((16,), jnp.float32)
                                for h in range(H):
                                    sh = jnp.zeros((16,), jnp.float32)
                                    for k in range(4):
                                        col = h * C + k * 16
                                        t = (rows[r, pl.ds(col, 16)]
                                             + rowd[r, pl.ds(col, 16)]
                                             + ear * wev_v[pl.ds(col, 16)])
                                        t = jnp.maximum(t, 0.2 * t)
                                        sh = sh + attv[pl.ds(col, 16)] * t
                                    lg = jnp.sum(sh)
                                    wv = wv + jnp.where(iota == h, _f16(lg), 0.0)
                                wbuf[r, pl.ds(0, 16)] = jnp.exp(wv)
                                return 0
                            lax.fori_loop(0, kb, wcomp, 0)
                        if kind in ("gat", "gatv2"):
                            def scale(r, _):
                                for h in range(H):
                                    wh = plsc.load_gather(
                                        wbuf, [_f16(r), _f16(jnp.int32(h))])
                                    for k in range(4):
                                        col = h * C + k * 16
                                        rows[r, pl.ds(col, 16)] = (
                                            rows[r, pl.ds(col, 16)] * wh)
                                return 0
                            lax.fori_loop(0, kb, scale, 0)

                        pltpu.sync_copy(rows, acc.at[ldst], add=True)
                        pltpu.sync_copy(wbuf, sacc.at[ldst], add=True)
                        return 0
                    lax.fori_loop(0, (cnt + kb - 1) // kb, blk, 0)
                plsc.subcore_barrier()
                pltpu.sync_copy(
                    acc.at[pl.ds(tid * nstripe_o, nstripe_o)],
                    sum_h.at[pl.ds(cc * SCH + tid * nstripe_o, nstripe_o)])
                pltpu.sync_copy(
                    sacc.at[pl.ds(tid * nstripe_o, nstripe_o)],
                    ws_h.at[pl.ds(cc * SCH + tid * nstripe_o, nstripe_o)])
                plsc.subcore_barrier()

    ins = [bk["src"], bk["dst"]]
    if kind == "gatv2":
        ins.append(bk["ea"])
    ins.append(bk["cnts"])
    ins.append(table)
    ins.extend(extra)

    out_type = (jax.ShapeDtypeStruct((nb * SCH, f), jnp.float32),
                jax.ShapeDtypeStruct((nb * SCH, 16), jnp.float32))

    scratch = [
        pltpu.VMEM_SHARED((ACC_R, f), jnp.float32),
        pltpu.VMEM_SHARED((ACC_R, 16), jnp.float32),
        pltpu.VMEM((kb,), jnp.int32), pltpu.VMEM((kb,), jnp.int32),
        pltpu.VMEM((kb,), jnp.int32),
        pltpu.VMEM((kb, f), jnp.float32), pltpu.VMEM((kb, 16), jnp.float32),
        pltpu.VMEM((kb, f), jnp.float32), pltpu.VMEM((kb, 16), jnp.float32),
        pltpu.VMEM((16,), jnp.int32),
    ]
    if kind == "gat":
        scratch += [pltpu.VMEM((kb, 16), jnp.float32),
                    pltpu.VMEM((kb, 16), jnp.float32)]
    if kind == "gatv2":
        scratch += [pltpu.VMEM((kb, f), jnp.float32),
                    pltpu.VMEM((kb,), jnp.float32),
                    pltpu.VMEM((f,), jnp.float32),
                    pltpu.VMEM((f,), jnp.float32)]
    scratch.append(pltpu.SemaphoreType.DMA)

    return pl.kernel(body, out_type=out_type, mesh=_mesh(),
                     scratch_types=tuple(scratch),
                     compiler_params=_SC_PARAMS)(*ins)


def _gat(x_src, x_dst, bk, p, n_dst):
    hs = _mm(x_src, p['W_src'])
    hd = _mm(x_dst, p['W_dst'])
    a_s = (hs.reshape(-1, H, C) * p['att_src']).sum(-1)
    a_d = (hd.reshape(-1, H, C) * p['att_dst']).sum(-1)
    asr = jnp.pad(a_s, ((0, 0), (0, 16 - H)))
    adr = jnp.pad(a_d, ((0, 0), (0, 16 - H)))
    num, s = _edge_agg("gat", bk, hs, H * C, extra=(asr, adr))
    num = num[:n_dst]
    s = s[:n_dst, :H]
    w_self = jnp.exp(jax.nn.leaky_relu(a_s + a_d, 0.2))
    s = s + w_self
    num = num + hs * jnp.repeat(w_self, C, axis=1)
    out = num / (jnp.repeat(s, C, axis=1) + 1e-16)
    return out.reshape(-1, H, C).mean(axis=1) + p['b']


def _gatv2(x_src, x_dst, bk, ea, p, n_dst):
    hs = _mm(x_src, p['W_l']) + p['b_l']
    hd = _mm(x_dst, p['W_r']) + p['b_r']
    attv = p['att'].reshape(H * C)
    wev = p['W_e'].reshape(H * C)
    num, s = _edge_agg("gatv2", bk, hs, H * C, extra=(hd, attv, wev))
    num = num[:n_dst]
    s = s[:n_dst, :H]
    he_loop = jnp.mean(ea) * wev
    e_loop = jax.nn.leaky_relu(hs + hd + he_loop, 0.2)
    lg_loop = (e_loop.reshape(-1, H, C) * p['att']).sum(-1)
    w_self = jnp.exp(lg_loop)
    s = s + w_self
    num = num + hs * jnp.repeat(w_self, C, axis=1)
    out = num / (jnp.repeat(s, C, axis=1) + 1e-16)
    return out.reshape(-1, H, C).mean(axis=1) + p['b']


def _sage(x_src, x_dst, bk, p, n_dst):
    xs = jax.nn.relu(_mm(x_src, p['Wp']) + p['bp'])
    f = xs.shape[1]
    ssum, scnt = _edge_agg("sage", bk, xs, f)
    cnt = scnt[:n_dst, 0]
    aggr = ssum[:n_dst] / jnp.maximum(cnt, 1.0)[:, None]
    out = _mm(aggr, p['W_l']) + p['b_l'] + _mm(x_dst, p['W_r'])
    return out / jnp.maximum(jnp.linalg.norm(out, axis=-1, keepdims=True), 1e-12)


def _hetero_layer(xa, xt, xg, B, ea_lf, P):
    Na, Nt, Ng = xa.shape[0], xt.shape[0], xg.shape[0]
    outs_a = [
        _gat(xa, xa, B['collab'], P['collab'], Na),
        _gatv2(xa, xa, B['lastfm'], ea_lf, P['lastfm'], Na),
        _gat(xa, xa, B['linked'], P['linked'], Na),
        _gat(xa, xa, B['musical'], P['musical'], Na),
        _gat(xa, xa, B['personal'], P['personal'], Na),
        _sage(xg, xa, B['tags_artists'], P['tags_artists'], Na),
        _sage(xt, xa, B['worked_by'], P['worked_by'], Na),
    ]
    outs_g = [
        _sage(xa, xg, B['has_tag_artists'], P['has_tag_artists'], Ng),
        _sage(xt, xg, B['has_tag_tracks'], P['has_tag_tracks'], Ng),
    ]
    outs_t = [
        _sage(xg, xt, B['tags_tracks'], P['tags_tracks'], Nt),
        _sage(xa, xt, B['worked_in'], P['worked_in'], Nt),
    ]
    return (jnp.mean(jnp.stack(outs_a), 0), jnp.mean(jnp.stack(outs_t), 0),
            jnp.mean(jnp.stack(outs_g), 0))


def kernel(x_artist, x_track, x_tag, edge_attr_lastfm, params,
           edge_index_collab, edge_index_lastfm, edge_index_linked,
           edge_index_musical, edge_index_personal,
           edge_index_has_tag_artists, edge_index_has_tag_tracks,
           edge_index_tags_artists, edge_index_tags_tracks,
           edge_index_worked_by, edge_index_worked_in):
    Na, Nt, Ng = x_artist.shape[0], x_track.shape[0], x_tag.shape[0]
    B = {
        'collab': _bucket(edge_index_collab, Na),
        'lastfm': _bucket(edge_index_lastfm, Na, ea=edge_attr_lastfm),
        'linked': _bucket(edge_index_linked, Na),
        'musical': _bucket(edge_index_musical, Na),
        'personal': _bucket(edge_index_personal, Na),
        'has_tag_artists': _bucket(edge_index_has_tag_artists, Ng),
        'has_tag_tracks': _bucket(edge_index_has_tag_tracks, Ng),
        'tags_artists': _bucket(edge_index_tags_artists, Na),
        'tags_tracks': _bucket(edge_index_tags_tracks, Nt),
        'worked_by': _bucket(edge_index_worked_by, Na),
        'worked_in': _bucket(edge_index_worked_in, Nt),
    }
    a1, t1, g1 = _hetero_layer(x_artist, x_track, x_tag, B, edge_attr_lastfm,
                               params['l1'])
    a2, t2, g2 = _hetero_layer(a1, t1, g1, B, edge_attr_lastfm, params['l2'])
    x = jnp.concatenate([a1, a2], axis=-1)
    x = jax.nn.relu(_mm(x, params['lin1_W']) + params['lin1_b'])
    x = _mm(x, params['lin2_W']) + params['lin2_b']
    return x / jnp.maximum(jnp.linalg.norm(x, axis=-1, keepdims=True), 1e-12)
